# baseline (speedup 1.0000x reference)
import functools

import jax
import jax.numpy as jnp
from jax import lax
from jax.experimental import pallas as pl
from jax.experimental.pallas import tpu as pltpu
from jax.experimental.pallas import tpu_sc as plsc

B, L = 1024, 200
S = 8
C = 16
TOK = B * L
ROWS = TOK // S
D_OUT = 128


_NW = 32
_CHUNK = 128
_TPW = TOK // _NW
_CPW = _TPW // _CHUNK
_PC = 5
_NPASS = _CPW // _PC
_WPP = _PC * _CHUNK * C
_WPW = _TPW * C


def _sc_body(tv, td, t0, t1, t2, iv, id_, i0, i1, i2, out,
             bv, bd, b0, b1, b2, gv, gd, g0, g1, g2, packed, sem):
    wid = lax.axis_index("s") * 2 + lax.axis_index("c")
    base = wid * _TPW
    idx_bufs = (bv, bd, b0, b1, b2)
    for ihbm, ivm in zip((iv, id_, i0, i1, i2), idx_bufs):
        pltpu.sync_copy(ihbm.at[pl.ds(base, _TPW)], ivm)
    tabs = (tv, td, t0, t1, t2)
    gbufs = (gv, gd, g0, g1, g2)
    for p in range(_NPASS):
        descs = []
        for tab, ivm, gbuf in zip(tabs, idx_bufs, gbufs):
            for ci in range(_PC):
                off = (p * _PC + ci) * _CHUNK
                descs.append(pltpu.async_copy(
                    tab.at[ivm.at[pl.ds(off, _CHUNK)]], gbuf.at[ci], sem))
        for d in descs:
            d.wait()

        for ci in range(_PC):
            pbase = ci * (_CHUNK * C)

            def row(r, _, _ci=ci, _pb=pbase):
                v = (gv[_ci, r] + gd[_ci, r] + g0[_ci, r]
                     + g1[_ci, r] + g2[_ci, r])
                packed[pl.ds(_pb + r * C, C)] = v
                return 0

            lax.fori_loop(0, _CHUNK, row, 0)
        pltpu.sync_copy(packed, out.at[pl.ds(wid * _WPW + p * _WPP, _WPP)])


def _sc_gather_sum(tv, td, t0, t1, t2, iv, id_, i0, i1, i2):
    mesh = plsc.VectorSubcoreMesh(core_axis_name="c", subcore_axis_name="s")
    kern = functools.partial(
        pl.kernel,
        mesh=mesh,
        compiler_params=pltpu.CompilerParams(use_tc_tiling_on_sc=False),
        out_type=jax.ShapeDtypeStruct((TOK * C,), jnp.float32),
        scratch_types=(
            [pltpu.VMEM((_TPW,), jnp.int32) for _ in range(5)]
            + [pltpu.VMEM((_PC, _CHUNK, C), jnp.float32) for _ in range(5)]
            + [pltpu.VMEM((_WPP,), jnp.float32), pltpu.SemaphoreType.DMA]
        ),
    )(_sc_body)
    return kern(tv, td, t0, t1, t2, iv, id_, i0, i1, i2)


_GRID = 32
_RB = ROWS // _GRID


def _tc_body(xv_ref, vid_ref, e8_ref, wt_ref, b_ref, out_ref):
    hi = jax.lax.Precision.HIGHEST
    mvec = (vid_ref[...] != 0).astype(jnp.float32)
    mask = jax.lax.dot(mvec, e8_ref[...], precision=hi)
    x = xv_ref[...] * mask
    out_ref[...] = jax.lax.dot(x, wt_ref[...], precision=hi) + b_ref[...]


def _tc_mask_conv(xv, vid, e8, wt, bias):
    def full(shape):
        return pl.BlockSpec(shape, lambda *_: tuple(0 for _ in shape))

    return pl.pallas_call(
        _tc_body,
        grid=(_GRID,),
        in_specs=[
            pl.BlockSpec((_RB, D_OUT), lambda i: (i, 0)),
            pl.BlockSpec((_RB, S), lambda i: (i, 0)),
            full((S, D_OUT)),
            full((S * C, D_OUT)),
            full((1, D_OUT)),
        ],
        out_specs=pl.BlockSpec((_RB, D_OUT), lambda i: (i, 0)),
        out_shape=jax.ShapeDtypeStruct((ROWS, D_OUT), jnp.float32),
    )(xv, vid, e8, wt, bias)


def kernel(value, depth, position, tgt_value_emb, tgt_depth_emb,
           tgt_spatial_emb, conv_w, conv_b):
    value = value.astype(jnp.int32)
    depth = depth.astype(jnp.int32)
    position = position.astype(jnp.int32)

    de8 = jnp.zeros((S, C), jnp.float32).at[1:7].set(tgt_depth_emb[1:])
    se_z = tgt_spatial_emb.at[:, 0, :].set(0.0)

    iv = value.reshape(TOK)
    id_ = depth.reshape(TOK)
    i0 = position[:, :, 0].reshape(TOK)
    i1 = position[:, :, 1].reshape(TOK)
    i2 = position[:, :, 2].reshape(TOK)

    xsum = _sc_gather_sum(tgt_value_emb, de8, se_z[0], se_z[1], se_z[2],
                          iv, id_, i0, i1, i2)
    xv = xsum.reshape(ROWS, D_OUT)

    vid = value.reshape(ROWS, S)
    e8 = jnp.repeat(jnp.eye(S, dtype=jnp.float32), C, axis=1)
    wt = conv_w.transpose(2, 1, 0).reshape(S * C, D_OUT)
    bias = conv_b.reshape(1, D_OUT)

    out = _tc_mask_conv(xv, vid, e8, wt, bias)
    return out.reshape(B, L // S, D_OUT)
